# baseline (speedup 1.0000x reference)
import functools

import jax
import jax.numpy as jnp
from jax import lax
from jax.experimental import pallas as pl
from jax.experimental.pallas import tpu as pltpu
from jax.experimental.pallas import tpu_sc as plsc

N = 50000
D = 64
E = 800000
HALF = 32
NS = 16
NC = 2

CH = 320
EPAD = 819200
EPT = EPAD // NS
NCH = EPT // CH
NPAD = 50176
RPT = NPAD // NS
NDUMMY = 128
ZROWS = CH
NLAYERS = 3


def _wb_chunks(chunk, total):
    return tuple((o, min(chunk, total - o)) for o in range(0, total, chunk))


_WB = _wb_chunks(CH, RPT)

_mesh = functools.partial(
    plsc.VectorSubcoreMesh, core_axis_name="c", subcore_axis_name="s")

_SC_PARAMS = pltpu.CompilerParams(use_tc_tiling_on_sc=False)


@functools.partial(
    pl.kernel,
    mesh=_mesh(),
    out_type=(
        jax.ShapeDtypeStruct((NLAYERS * NC * NPAD, HALF), jnp.float32),
        jax.ShapeDtypeStruct((NC * NPAD, HALF), jnp.float32),
    ),
    compiler_params=_SC_PARAMS,
    scratch_types=[
        pltpu.VMEM((CH,), jnp.int32),
        pltpu.VMEM((CH,), jnp.int32),
        pltpu.VMEM((CH,), jnp.int32),
        pltpu.VMEM((CH,), jnp.int32),
        pltpu.VMEM((CH, HALF), jnp.float32),
        pltpu.VMEM((CH, HALF), jnp.float32),
        pltpu.VMEM((CH,), jnp.float32),
        pltpu.VMEM((RPT,), jnp.float32),
        pltpu.VMEM_SHARED((NPAD, HALF), jnp.float32),
        pltpu.VMEM_SHARED((NPAD,), jnp.float32),
        pltpu.SemaphoreType.DMA,
        pltpu.SemaphoreType.DMA,
        pltpu.SemaphoreType.DMA,
        pltpu.SemaphoreType.DMA,
        pltpu.SemaphoreType.DMA,
        pltpu.SemaphoreType.DMA,
        pltpu.SemaphoreType.DMA,
        pltpu.SemaphoreType.DMA,
    ],
)
def _gcn_call(x0_hbm, src_hbm, dst_hbm, out_hbm, y_hbm,
              srcb0, srcb1, dstb0, dstb1, msg0, msg1, onesb, dinvb,
              acc, deg, isem0, isem1, gsem0, gsem1, ssem0, ssem1,
              zsem, ysem):
    c = lax.axis_index("c")
    s = lax.axis_index("s")
    srcb = (srcb0, srcb1)
    dstb = (dstb0, dstb1)
    msg = (msg0, msg1)
    isem = (isem0, isem1)
    gsem = (gsem0, gsem1)
    ssem = (ssem0, ssem1)
    def _fill(buf, val):
        def body(i, _):
            buf[pl.ds(i * 16, 16)] = jnp.full((16,), val, jnp.float32)
            return 0
        lax.fori_loop(0, CH // 16, body, 0)

    _fill(onesb, 0.0)
    for off, sz in _WB:
        pltpu.async_copy(x0_hbm.at[pl.ds(NC * NPAD, sz)],
                         acc.at[pl.ds(s * RPT + off, sz)], zsem)
        pltpu.async_copy(onesb.at[pl.ds(0, sz)],
                         deg.at[pl.ds(s * RPT + off, sz)], ysem)
    for off, sz in _WB:
        pltpu.make_async_copy(x0_hbm.at[pl.ds(NC * NPAD, sz)],
                              acc.at[pl.ds(s * RPT + off, sz)], zsem).wait()
        pltpu.make_async_copy(onesb.at[pl.ds(0, sz)],
                              deg.at[pl.ds(s * RPT + off, sz)], ysem).wait()
    _fill(onesb, 1.0)
    plsc.subcore_barrier()

    def _deg_front(i, b, wait_prev):
        if wait_prev:
            pltpu.make_async_copy(onesb, deg.at[dstb[b]], ssem[b]).wait()
        pltpu.async_copy(dst_hbm.at[pl.ds(s * EPT + i * CH, CH)],
                         dstb[b], isem[b]).wait()
        pltpu.async_copy(onesb, deg.at[dstb[b]], ssem[b], add=True)

    _deg_front(0, 0, False)
    _deg_front(1, 1, False)

    def _deg_pair(g, _):
        _deg_front(2 * g, 0, True)
        _deg_front(2 * g + 1, 1, True)
        return 0

    lax.fori_loop(1, NCH // 2, _deg_pair, 0)
    pltpu.make_async_copy(onesb, deg.at[dstb0], ssem0).wait()
    pltpu.make_async_copy(onesb, deg.at[dstb1], ssem1).wait()
    plsc.subcore_barrier()

    pltpu.sync_copy(deg.at[pl.ds(s * RPT, RPT)], dinvb)

    def _rsqrt16(i, _):
        v = dinvb[pl.ds(i * 16, 16)]
        bits = lax.bitcast_convert_type(v, jnp.int32)
        g = lax.bitcast_convert_type(
            jnp.int32(0x5F3759DF) - lax.shift_right_logical(bits, 1),
            jnp.float32)
        h = v * 0.5
        g = g * (1.5 - h * g * g)
        g = g * (1.5 - h * g * g)
        g = g * (1.5 - h * g * g)
        dinvb[pl.ds(i * 16, 16)] = jnp.where(v > 0, g, 0.0)
        return 0

    lax.fori_loop(0, RPT // 16, _rsqrt16, 0)

    def _scale_rows(buf_in, buf_out, row0, nrows, sq):
        def body(i, _):
            dvec = dinvb[pl.ds(row0 + i * 16, 16)]
            if sq:
                dvec = dvec * dvec
            for j in range(16):
                r = i * 16 + j
                dv = jnp.full((16,), dvec[j], jnp.float32)
                buf_out[r, pl.ds(0, 16)] = buf_in[r, pl.ds(0, 16)] * dv
                buf_out[r, pl.ds(16, 16)] = buf_in[r, pl.ds(16, 16)] * dv
            return 0
        lax.fori_loop(0, nrows // 16, body, 0)

    for off, sz in _WB:
        r0 = s * RPT + off
        pltpu.async_copy(x0_hbm.at[pl.ds(c * NPAD + r0, sz)],
                         msg0.at[pl.ds(0, sz)], gsem0).wait()
        _scale_rows(msg0, msg1, off, sz, False)
        pltpu.async_copy(msg1.at[pl.ds(0, sz)],
                         y_hbm.at[pl.ds(c * NPAD + r0, sz)], ysem).wait()
    plsc.subcore_barrier()

    def _issue_front(i, b, wait_prev_scatter):
        if wait_prev_scatter:
            pltpu.make_async_copy(msg[b], acc.at[dstb[b]], ssem[b]).wait()
        base = s * EPT + i * CH
        c1 = pltpu.async_copy(src_hbm.at[pl.ds(c * EPAD + base, CH)],
                              srcb[b], isem[b])
        c2 = pltpu.async_copy(dst_hbm.at[pl.ds(base, CH)], dstb[b], isem[b])
        c1.wait()
        c2.wait()
        pltpu.async_copy(y_hbm.at[srcb[b]], msg[b], gsem[b])

    def _issue_back(b):
        pltpu.make_async_copy(y_hbm.at[srcb[b]], msg[b], gsem[b]).wait()
        pltpu.async_copy(msg[b], acc.at[dstb[b]], ssem[b], add=True)

    for layer in range(NLAYERS):
        _issue_front(0, 0, False)
        _issue_front(1, 1, False)
        _issue_back(0)
        _issue_back(1)

        def _pair(g, _):
            _issue_front(2 * g, 0, True)
            _issue_front(2 * g + 1, 1, True)
            _issue_back(0)
            _issue_back(1)
            return 0

        lax.fori_loop(1, NCH // 2, _pair, 0)
        pltpu.make_async_copy(msg0, acc.at[dstb0], ssem0).wait()
        pltpu.make_async_copy(msg1, acc.at[dstb1], ssem1).wait()
        plsc.subcore_barrier()

        last = layer == NLAYERS - 1
        for k, (off, sz) in enumerate(_WB):
            r0 = s * RPT + off
            pltpu.async_copy(acc.at[pl.ds(r0, sz)], msg0.at[pl.ds(0, sz)],
                             gsem0).wait()
            if not last:
                pltpu.async_copy(x0_hbm.at[pl.ds(NC * NPAD, sz)],
                                 acc.at[pl.ds(r0, sz)], zsem)
                if k > 0:
                    poff, psz = _WB[k - 1]
                    pltpu.make_async_copy(
                        msg1.at[pl.ds(0, psz)],
                        y_hbm.at[pl.ds(c * NPAD + s * RPT + poff, psz)],
                        ysem).wait()
                _scale_rows(msg0, msg1, off, sz, True)
                pltpu.async_copy(
                    msg1.at[pl.ds(0, sz)],
                    y_hbm.at[pl.ds(c * NPAD + r0, sz)], ysem)
            _scale_rows(msg0, msg0, off, sz, False)
            pltpu.async_copy(
                msg0.at[pl.ds(0, sz)],
                out_hbm.at[pl.ds((layer * NC + c) * NPAD + r0, sz)],
                ssem0).wait()
        if not last:
            loff, lsz = _WB[-1]
            pltpu.make_async_copy(
                msg1.at[pl.ds(0, lsz)],
                y_hbm.at[pl.ds(c * NPAD + s * RPT + loff, lsz)], ysem).wait()
            for off, sz in _WB:
                r0 = s * RPT + off
                pltpu.make_async_copy(x0_hbm.at[pl.ds(NC * NPAD, sz)],
                                      acc.at[pl.ds(r0, sz)], zsem).wait()
            plsc.subcore_barrier()


def kernel(edge_index, edge_attrs, table):
    del edge_attrs
    src = edge_index[0]
    dst = edge_index[1]

    pad_i = jnp.arange(EPAD - E, dtype=jnp.int32)
    src_p = jnp.concatenate([src, pad_i % N])
    dst_p = jnp.concatenate([dst, N + pad_i % NDUMMY])
    src2 = jnp.concatenate([src_p, src_p + NPAD])

    t2 = table.reshape(N, NC, HALF).transpose(1, 0, 2)
    zfill = jnp.zeros((NPAD - N, HALF), jnp.float32)
    x0 = jnp.concatenate(
        [t2[0], zfill, t2[1], zfill, jnp.zeros((ZROWS, HALF), jnp.float32)])

    out_x, _ = _gcn_call(x0, src2, dst_p)
    xs = out_x.reshape(NLAYERS, NC, NPAD, HALF)[:, :, :N, :]
    out = ((t2 + xs.sum(axis=0)) * 0.25).transpose(1, 0, 2).reshape(N, D)
    return (table, out)


# speedup vs baseline: 1.0057x; 1.0057x over previous
import functools

import jax
import jax.numpy as jnp
from jax import lax
from jax.experimental import pallas as pl
from jax.experimental.pallas import tpu as pltpu
from jax.experimental.pallas import tpu_sc as plsc

N = 50000
D = 64
E = 800000
HALF = 32
NS = 16
NC = 2

CH = 320
EPAD = 819200
EPT = EPAD // NS
NCH = EPT // CH
NPAD = 50176
RPT = NPAD // NS
NDUMMY = 128
ZROWS = CH
NLAYERS = 3


def _wb_chunks(chunk, total):
    return tuple((o, min(chunk, total - o)) for o in range(0, total, chunk))


_WB = _wb_chunks(CH, RPT)

_mesh = functools.partial(
    plsc.VectorSubcoreMesh, core_axis_name="c", subcore_axis_name="s")

_SC_PARAMS = pltpu.CompilerParams(use_tc_tiling_on_sc=False)


@functools.partial(
    pl.kernel,
    mesh=_mesh(),
    out_type=(
        jax.ShapeDtypeStruct((NLAYERS * NC * NPAD, HALF), jnp.float32),
        jax.ShapeDtypeStruct((NC * NPAD, HALF), jnp.float32),
        jax.ShapeDtypeStruct((NC * NPAD,), jnp.float32),
    ),
    compiler_params=_SC_PARAMS,
    scratch_types=[
        pltpu.VMEM((CH,), jnp.int32),
        pltpu.VMEM((CH,), jnp.int32),
        pltpu.VMEM((CH,), jnp.int32),
        pltpu.VMEM((CH,), jnp.int32),
        pltpu.VMEM((CH, HALF), jnp.float32),
        pltpu.VMEM((CH, HALF), jnp.float32),
        pltpu.VMEM((CH,), jnp.float32),
        pltpu.VMEM((RPT,), jnp.float32),
        pltpu.VMEM_SHARED((NPAD, HALF), jnp.float32),
        pltpu.VMEM_SHARED((NPAD,), jnp.float32),
        pltpu.SemaphoreType.DMA,
        pltpu.SemaphoreType.DMA,
        pltpu.SemaphoreType.DMA,
        pltpu.SemaphoreType.DMA,
        pltpu.SemaphoreType.DMA,
        pltpu.SemaphoreType.DMA,
        pltpu.SemaphoreType.DMA,
        pltpu.SemaphoreType.DMA,
    ],
)
def _gcn_call(x0_hbm, src_hbm, dst_hbm, out_hbm, y_hbm, dinv_hbm,
              srcb0, srcb1, dstb0, dstb1, msg0, msg1, onesb, dinvb,
              acc, deg, isem0, isem1, gsem0, gsem1, ssem0, ssem1,
              zsem, ysem):
    c = lax.axis_index("c")
    s = lax.axis_index("s")
    srcb = (srcb0, srcb1)
    dstb = (dstb0, dstb1)
    msg = (msg0, msg1)
    isem = (isem0, isem1)
    gsem = (gsem0, gsem1)
    ssem = (ssem0, ssem1)
    def _fill(buf, val):
        def body(i, _):
            buf[pl.ds(i * 16, 16)] = jnp.full((16,), val, jnp.float32)
            return 0
        lax.fori_loop(0, CH // 16, body, 0)

    _fill(onesb, 0.0)
    for off, sz in _WB:
        pltpu.async_copy(x0_hbm.at[pl.ds(NC * NPAD, sz)],
                         acc.at[pl.ds(s * RPT + off, sz)], zsem)
        pltpu.async_copy(onesb.at[pl.ds(0, sz)],
                         deg.at[pl.ds(s * RPT + off, sz)], ysem)
    for off, sz in _WB:
        pltpu.make_async_copy(x0_hbm.at[pl.ds(NC * NPAD, sz)],
                              acc.at[pl.ds(s * RPT + off, sz)], zsem).wait()
        pltpu.make_async_copy(onesb.at[pl.ds(0, sz)],
                              deg.at[pl.ds(s * RPT + off, sz)], ysem).wait()
    _fill(onesb, 1.0)
    plsc.subcore_barrier()

    def _deg_front(i, b, wait_prev):
        if wait_prev:
            pltpu.make_async_copy(onesb, deg.at[dstb[b]], ssem[b]).wait()
        pltpu.async_copy(dst_hbm.at[pl.ds(s * EPT + i * CH, CH)],
                         dstb[b], isem[b]).wait()
        pltpu.async_copy(onesb, deg.at[dstb[b]], ssem[b], add=True)

    _deg_front(0, 0, False)
    _deg_front(1, 1, False)

    def _deg_pair(g, _):
        _deg_front(2 * g, 0, True)
        _deg_front(2 * g + 1, 1, True)
        return 0

    lax.fori_loop(1, NCH // 2, _deg_pair, 0)
    pltpu.make_async_copy(onesb, deg.at[dstb0], ssem0).wait()
    pltpu.make_async_copy(onesb, deg.at[dstb1], ssem1).wait()
    plsc.subcore_barrier()

    pltpu.sync_copy(deg.at[pl.ds(s * RPT, RPT)], dinvb)

    def _rsqrt16(i, _):
        v = dinvb[pl.ds(i * 16, 16)]
        bits = lax.bitcast_convert_type(v, jnp.int32)
        g = lax.bitcast_convert_type(
            jnp.int32(0x5F3759DF) - lax.shift_right_logical(bits, 1),
            jnp.float32)
        h = v * 0.5
        g = g * (1.5 - h * g * g)
        g = g * (1.5 - h * g * g)
        g = g * (1.5 - h * g * g)
        dinvb[pl.ds(i * 16, 16)] = jnp.where(v > 0, g, 0.0)
        return 0

    lax.fori_loop(0, RPT // 16, _rsqrt16, 0)
    pltpu.async_copy(dinvb, dinv_hbm.at[pl.ds(c * NPAD + s * RPT, RPT)],
                     ysem).wait()

    def _scale_rows(buf_in, buf_out, row0, nrows, sq):
        def body(i, _):
            dvec = dinvb[pl.ds(row0 + i * 16, 16)]
            if sq:
                dvec = dvec * dvec
            for j in range(16):
                r = i * 16 + j
                dv = jnp.full((16,), dvec[j], jnp.float32)
                buf_out[r, pl.ds(0, 16)] = buf_in[r, pl.ds(0, 16)] * dv
                buf_out[r, pl.ds(16, 16)] = buf_in[r, pl.ds(16, 16)] * dv
            return 0
        lax.fori_loop(0, nrows // 16, body, 0)

    for off, sz in _WB:
        r0 = s * RPT + off
        pltpu.async_copy(x0_hbm.at[pl.ds(c * NPAD + r0, sz)],
                         msg0.at[pl.ds(0, sz)], gsem0).wait()
        _scale_rows(msg0, msg1, off, sz, False)
        pltpu.async_copy(msg1.at[pl.ds(0, sz)],
                         y_hbm.at[pl.ds(c * NPAD + r0, sz)], ysem).wait()
    plsc.subcore_barrier()

    def _issue_front(i, b, wait_prev_scatter):
        if wait_prev_scatter:
            pltpu.make_async_copy(msg[b], acc.at[dstb[b]], ssem[b]).wait()
        base = s * EPT + i * CH
        c1 = pltpu.async_copy(src_hbm.at[pl.ds(c * EPAD + base, CH)],
                              srcb[b], isem[b])
        c2 = pltpu.async_copy(dst_hbm.at[pl.ds(base, CH)], dstb[b], isem[b])
        c1.wait()
        c2.wait()
        pltpu.async_copy(y_hbm.at[srcb[b]], msg[b], gsem[b])

    def _issue_back(b):
        pltpu.make_async_copy(y_hbm.at[srcb[b]], msg[b], gsem[b]).wait()
        pltpu.async_copy(msg[b], acc.at[dstb[b]], ssem[b], add=True)

    for layer in range(NLAYERS):
        _issue_front(0, 0, False)
        _issue_front(1, 1, False)
        _issue_back(0)
        _issue_back(1)

        def _pair(g, _):
            _issue_front(2 * g, 0, True)
            _issue_front(2 * g + 1, 1, True)
            _issue_back(0)
            _issue_back(1)
            return 0

        lax.fori_loop(1, NCH // 2, _pair, 0)
        pltpu.make_async_copy(msg0, acc.at[dstb0], ssem0).wait()
        pltpu.make_async_copy(msg1, acc.at[dstb1], ssem1).wait()
        plsc.subcore_barrier()

        last = layer == NLAYERS - 1
        if last:
            for off, sz in _WB:
                r0 = s * RPT + off
                pltpu.async_copy(
                    acc.at[pl.ds(r0, sz)],
                    out_hbm.at[pl.ds((layer * NC + c) * NPAD + r0, sz)],
                    ssem0)
            for off, sz in _WB:
                r0 = s * RPT + off
                pltpu.make_async_copy(
                    acc.at[pl.ds(r0, sz)],
                    out_hbm.at[pl.ds((layer * NC + c) * NPAD + r0, sz)],
                    ssem0).wait()
        else:
            for k, (off, sz) in enumerate(_WB):
                r0 = s * RPT + off
                pltpu.async_copy(acc.at[pl.ds(r0, sz)], msg0.at[pl.ds(0, sz)],
                                 gsem0).wait()
                pltpu.async_copy(x0_hbm.at[pl.ds(NC * NPAD, sz)],
                                 acc.at[pl.ds(r0, sz)], zsem)
                if k > 0:
                    poff, psz = _WB[k - 1]
                    pltpu.make_async_copy(
                        msg1.at[pl.ds(0, psz)],
                        y_hbm.at[pl.ds(c * NPAD + s * RPT + poff, psz)],
                        ysem).wait()
                _scale_rows(msg0, msg1, off, sz, True)
                pltpu.async_copy(
                    msg1.at[pl.ds(0, sz)],
                    y_hbm.at[pl.ds(c * NPAD + r0, sz)], ysem)
                pltpu.async_copy(
                    msg0.at[pl.ds(0, sz)],
                    out_hbm.at[pl.ds((layer * NC + c) * NPAD + r0, sz)],
                    ssem0).wait()
            loff, lsz = _WB[-1]
            pltpu.make_async_copy(
                msg1.at[pl.ds(0, lsz)],
                y_hbm.at[pl.ds(c * NPAD + s * RPT + loff, lsz)], ysem).wait()
            for off, sz in _WB:
                r0 = s * RPT + off
                pltpu.make_async_copy(x0_hbm.at[pl.ds(NC * NPAD, sz)],
                                      acc.at[pl.ds(r0, sz)], zsem).wait()
            plsc.subcore_barrier()


def kernel(edge_index, edge_attrs, table):
    del edge_attrs
    src = edge_index[0]
    dst = edge_index[1]

    pad_i = jnp.arange(EPAD - E, dtype=jnp.int32)
    src_p = jnp.concatenate([src, pad_i % N])
    dst_p = jnp.concatenate([dst, N + pad_i % NDUMMY])
    src2 = jnp.concatenate([src_p, src_p + NPAD])

    t2 = table.reshape(N, NC, HALF).transpose(1, 0, 2)
    zfill = jnp.zeros((NPAD - N, HALF), jnp.float32)
    x0 = jnp.concatenate(
        [t2[0], zfill, t2[1], zfill, jnp.zeros((ZROWS, HALF), jnp.float32)])

    out_raw, _, dinv = _gcn_call(x0, src2, dst_p)
    raws = out_raw.reshape(NLAYERS, NC, NPAD, HALF)[:, :, :N, :]
    d3 = dinv[:N][None, :, None]
    out = ((t2 + d3 * raws.sum(axis=0)) * 0.25).transpose(1, 0, 2)
    return (table, out.reshape(N, D))


# speedup vs baseline: 1.1763x; 1.1697x over previous
import functools

import jax
import jax.numpy as jnp
from jax import lax
from jax.experimental import pallas as pl
from jax.experimental.pallas import tpu as pltpu
from jax.experimental.pallas import tpu_sc as plsc

N = 50000
D = 64
E = 800000
HALF = 32
NS = 16
NC = 2

CH = 320
EPAD = 819200
EPT = EPAD // NS
NCH = EPT // CH
NPAD = 50176
RPT = NPAD // NS
NDUMMY = 128
ZROWS = CH
NLAYERS = 3


def _wb_chunks(chunk, total):
    return tuple((o, min(chunk, total - o)) for o in range(0, total, chunk))


_WB = _wb_chunks(CH, RPT)

_mesh = functools.partial(
    plsc.VectorSubcoreMesh, core_axis_name="c", subcore_axis_name="s")

_SC_PARAMS = pltpu.CompilerParams(use_tc_tiling_on_sc=False)


@functools.partial(
    pl.kernel,
    mesh=_mesh(),
    out_type=(
        jax.ShapeDtypeStruct((NPAD, D), jnp.float32),
        jax.ShapeDtypeStruct((NC * NPAD, HALF), jnp.float32),
    ),
    compiler_params=_SC_PARAMS,
    scratch_types=[
        pltpu.VMEM((CH,), jnp.int32),
        pltpu.VMEM((CH,), jnp.int32),
        pltpu.VMEM((CH,), jnp.int32),
        pltpu.VMEM((CH,), jnp.int32),
        pltpu.VMEM((CH, HALF), jnp.float32),
        pltpu.VMEM((CH, HALF), jnp.float32),
        pltpu.VMEM((CH,), jnp.float32),
        pltpu.VMEM((RPT,), jnp.float32),
        pltpu.VMEM_SHARED((NPAD, HALF), jnp.float32),
        pltpu.VMEM_SHARED((NPAD,), jnp.float32),
        pltpu.SemaphoreType.DMA,
        pltpu.SemaphoreType.DMA,
        pltpu.SemaphoreType.DMA,
        pltpu.SemaphoreType.DMA,
        pltpu.SemaphoreType.DMA,
        pltpu.SemaphoreType.DMA,
        pltpu.SemaphoreType.DMA,
        pltpu.SemaphoreType.DMA,
    ],
)
def _gcn_call(x0_hbm, src_hbm, dst_hbm, out_hbm, y_hbm,
              srcb0, srcb1, dstb0, dstb1, msg0, msg1, onesb, dinvb,
              acc, deg, isem0, isem1, gsem0, gsem1, ssem0, ssem1,
              zsem, ysem):
    c = lax.axis_index("c")
    s = lax.axis_index("s")
    srcb = (srcb0, srcb1)
    dstb = (dstb0, dstb1)
    msg = (msg0, msg1)
    isem = (isem0, isem1)
    gsem = (gsem0, gsem1)
    ssem = (ssem0, ssem1)
    def _fill(buf, val):
        def body(i, _):
            buf[pl.ds(i * 16, 16)] = jnp.full((16,), val, jnp.float32)
            return 0
        lax.fori_loop(0, CH // 16, body, 0)

    _fill(onesb, 0.0)
    for off, sz in _WB:
        pltpu.async_copy(x0_hbm.at[pl.ds(NC * NPAD, sz)],
                         acc.at[pl.ds(s * RPT + off, sz)], zsem)
        pltpu.async_copy(onesb.at[pl.ds(0, sz)],
                         deg.at[pl.ds(s * RPT + off, sz)], ysem)
    for off, sz in _WB:
        pltpu.make_async_copy(x0_hbm.at[pl.ds(NC * NPAD, sz)],
                              acc.at[pl.ds(s * RPT + off, sz)], zsem).wait()
        pltpu.make_async_copy(onesb.at[pl.ds(0, sz)],
                              deg.at[pl.ds(s * RPT + off, sz)], ysem).wait()
    _fill(onesb, 1.0)
    plsc.subcore_barrier()

    def _deg_front(i, b, wait_prev):
        if wait_prev:
            pltpu.make_async_copy(onesb, deg.at[dstb[b]], ssem[b]).wait()
        pltpu.async_copy(dst_hbm.at[pl.ds(s * EPT + i * CH, CH)],
                         dstb[b], isem[b]).wait()
        pltpu.async_copy(onesb, deg.at[dstb[b]], ssem[b], add=True)

    _deg_front(0, 0, False)
    _deg_front(1, 1, False)

    def _deg_pair(g, _):
        _deg_front(2 * g, 0, True)
        _deg_front(2 * g + 1, 1, True)
        return 0

    lax.fori_loop(1, NCH // 2, _deg_pair, 0)
    pltpu.make_async_copy(onesb, deg.at[dstb0], ssem0).wait()
    pltpu.make_async_copy(onesb, deg.at[dstb1], ssem1).wait()
    plsc.subcore_barrier()

    pltpu.sync_copy(deg.at[pl.ds(s * RPT, RPT)], dinvb)

    def _rsqrt16(i, _):
        v = dinvb[pl.ds(i * 16, 16)]
        bits = lax.bitcast_convert_type(v, jnp.int32)
        g = lax.bitcast_convert_type(
            jnp.int32(0x5F3759DF) - lax.shift_right_logical(bits, 1),
            jnp.float32)
        h = v * 0.5
        g = g * (1.5 - h * g * g)
        g = g * (1.5 - h * g * g)
        g = g * (1.5 - h * g * g)
        dinvb[pl.ds(i * 16, 16)] = jnp.where(v > 0, g, 0.0)
        return 0

    lax.fori_loop(0, RPT // 16, _rsqrt16, 0)

    def _scale_rows(buf_in, buf_out, row0, nrows, sq):
        def body(i, _):
            dvec = dinvb[pl.ds(row0 + i * 16, 16)]
            if sq:
                dvec = dvec * dvec
            for j in range(16):
                r = i * 16 + j
                dv = jnp.full((16,), dvec[j], jnp.float32)
                buf_out[r, pl.ds(0, 16)] = buf_in[r, pl.ds(0, 16)] * dv
                buf_out[r, pl.ds(16, 16)] = buf_in[r, pl.ds(16, 16)] * dv
            return 0
        lax.fori_loop(0, nrows // 16, body, 0)

    def _sub_scaled(row0, nrows):
        def body(i, _):
            dvec = dinvb[pl.ds(row0 + i * 16, 16)]
            dd = dvec * dvec
            for j in range(16):
                r = i * 16 + j
                dv = jnp.full((16,), dd[j], jnp.float32)
                msg0[r, pl.ds(0, 16)] = (
                    msg0[r, pl.ds(0, 16)] * dv - msg1[r, pl.ds(0, 16)])
                msg0[r, pl.ds(16, 16)] = (
                    msg0[r, pl.ds(16, 16)] * dv - msg1[r, pl.ds(16, 16)])
            return 0
        lax.fori_loop(0, nrows // 16, body, 0)

    def _final_rows(row0, nrows):
        def body(i, _):
            dvec = dinvb[pl.ds(row0 + i * 16, 16)] * 0.25
            for j in range(16):
                r = i * 16 + j
                dv = jnp.full((16,), dvec[j], jnp.float32)
                msg0[r, pl.ds(0, 16)] = (
                    msg1[r, pl.ds(0, 16)] * 0.25 + msg0[r, pl.ds(0, 16)] * dv)
                msg0[r, pl.ds(16, 16)] = (
                    msg1[r, pl.ds(16, 16)] * 0.25 + msg0[r, pl.ds(16, 16)] * dv)
            return 0
        lax.fori_loop(0, nrows // 16, body, 0)

    for off, sz in _WB:
        r0 = s * RPT + off
        pltpu.async_copy(x0_hbm.at[pl.ds(c * NPAD + r0, sz)],
                         msg0.at[pl.ds(0, sz)], gsem0).wait()
        _scale_rows(msg0, msg1, off, sz, False)
        pltpu.async_copy(msg1.at[pl.ds(0, sz)],
                         y_hbm.at[pl.ds(c * NPAD + r0, sz)], ysem).wait()
    plsc.subcore_barrier()

    def _issue_front(i, b, wait_prev_scatter):
        if wait_prev_scatter:
            pltpu.make_async_copy(msg[b], acc.at[dstb[b]], ssem[b]).wait()
        base = s * EPT + i * CH
        c1 = pltpu.async_copy(src_hbm.at[pl.ds(c * EPAD + base, CH)],
                              srcb[b], isem[b])
        c2 = pltpu.async_copy(dst_hbm.at[pl.ds(base, CH)], dstb[b], isem[b])
        c1.wait()
        c2.wait()
        pltpu.async_copy(y_hbm.at[srcb[b]], msg[b], gsem[b])

    def _issue_back(b):
        pltpu.make_async_copy(y_hbm.at[srcb[b]], msg[b], gsem[b]).wait()
        pltpu.async_copy(msg[b], acc.at[dstb[b]], ssem[b], add=True)

    for layer in range(NLAYERS):
        _issue_front(0, 0, False)
        _issue_front(1, 1, False)
        _issue_back(0)
        _issue_back(1)

        def _pair(g, _):
            _issue_front(2 * g, 0, True)
            _issue_front(2 * g + 1, 1, True)
            _issue_back(0)
            _issue_back(1)
            return 0

        lax.fori_loop(1, NCH // 2, _pair, 0)
        pltpu.make_async_copy(msg0, acc.at[dstb0], ssem0).wait()
        pltpu.make_async_copy(msg1, acc.at[dstb1], ssem1).wait()
        plsc.subcore_barrier()

        if layer == 0:
            for k, (off, sz) in enumerate(_WB):
                r0 = s * RPT + off
                pltpu.async_copy(acc.at[pl.ds(r0, sz)], msg0.at[pl.ds(0, sz)],
                                 gsem0).wait()
                if k > 0:
                    poff, psz = _WB[k - 1]
                    pltpu.make_async_copy(
                        msg1.at[pl.ds(0, psz)],
                        y_hbm.at[pl.ds(c * NPAD + s * RPT + poff, psz)],
                        ysem).wait()
                _scale_rows(msg0, msg1, off, sz, True)
                pltpu.async_copy(
                    msg1.at[pl.ds(0, sz)],
                    y_hbm.at[pl.ds(c * NPAD + r0, sz)], ysem)
            loff, lsz = _WB[-1]
            pltpu.make_async_copy(
                msg1.at[pl.ds(0, lsz)],
                y_hbm.at[pl.ds(c * NPAD + s * RPT + loff, lsz)], ysem).wait()
            plsc.subcore_barrier()
        elif layer == 1:
            for k, (off, sz) in enumerate(_WB):
                r0 = s * RPT + off
                if k > 0:
                    poff, psz = _WB[k - 1]
                    pltpu.make_async_copy(
                        msg0.at[pl.ds(0, psz)],
                        y_hbm.at[pl.ds(c * NPAD + s * RPT + poff, psz)],
                        ssem0).wait()
                ca = pltpu.async_copy(acc.at[pl.ds(r0, sz)],
                                      msg0.at[pl.ds(0, sz)], gsem0)
                cy = pltpu.async_copy(y_hbm.at[pl.ds(c * NPAD + r0, sz)],
                                      msg1.at[pl.ds(0, sz)], ysem)
                ca.wait()
                cy.wait()
                _sub_scaled(off, sz)
                pltpu.async_copy(
                    msg0.at[pl.ds(0, sz)],
                    y_hbm.at[pl.ds(c * NPAD + r0, sz)], ssem0)
            loff, lsz = _WB[-1]
            pltpu.make_async_copy(
                msg0.at[pl.ds(0, lsz)],
                y_hbm.at[pl.ds(c * NPAD + s * RPT + loff, lsz)], ssem0).wait()
            plsc.subcore_barrier()
        else:
            for k, (off, sz) in enumerate(_WB):
                r0 = s * RPT + off
                if k > 0:
                    poff, psz = _WB[k - 1]
                    pltpu.make_async_copy(
                        msg0.at[pl.ds(0, psz)],
                        out_hbm.at[pl.ds(s * RPT + poff, psz),
                                   pl.ds(c * HALF, HALF)], ssem0).wait()
                ca = pltpu.async_copy(acc.at[pl.ds(r0, sz)],
                                      msg0.at[pl.ds(0, sz)], gsem0)
                cx = pltpu.async_copy(x0_hbm.at[pl.ds(c * NPAD + r0, sz)],
                                      msg1.at[pl.ds(0, sz)], ysem)
                ca.wait()
                cx.wait()
                _final_rows(off, sz)
                pltpu.async_copy(
                    msg0.at[pl.ds(0, sz)],
                    out_hbm.at[pl.ds(r0, sz), pl.ds(c * HALF, HALF)], ssem0)
            loff, lsz = _WB[-1]
            pltpu.make_async_copy(
                msg0.at[pl.ds(0, lsz)],
                out_hbm.at[pl.ds(s * RPT + loff, lsz),
                           pl.ds(c * HALF, HALF)], ssem0).wait()


def kernel(edge_index, edge_attrs, table):
    del edge_attrs
    src = edge_index[0]
    dst = edge_index[1]

    pad_i = jnp.arange(EPAD - E, dtype=jnp.int32)
    src_p = jnp.concatenate([src, pad_i % N])
    dst_p = jnp.concatenate([dst, N + pad_i % NDUMMY])
    src2 = jnp.concatenate([src_p, src_p + NPAD])

    t2 = table.reshape(N, NC, HALF).transpose(1, 0, 2)
    zfill = jnp.zeros((NPAD - N, HALF), jnp.float32)
    x0 = jnp.concatenate(
        [t2[0], zfill, t2[1], zfill, jnp.zeros((ZROWS, HALF), jnp.float32)])

    final, _ = _gcn_call(x0, src2, dst_p)
    return (table, final[:N])


# speedup vs baseline: 1.2499x; 1.0625x over previous
import functools

import jax
import jax.numpy as jnp
from jax import lax
from jax.experimental import pallas as pl
from jax.experimental.pallas import tpu as pltpu
from jax.experimental.pallas import tpu_sc as plsc

N = 50000
D = 64
E = 800000
HALF = 32
NS = 16
NC = 2

CH = 256
EPAD = 819200
EPT = EPAD // NS
NCH = EPT // CH
NPAD = 50176
RPT = NPAD // NS
NDUMMY = 128
NLAYERS = 3


def _wb_chunks(chunk, total):
    return tuple((o, min(chunk, total - o)) for o in range(0, total, chunk))


_WB = _wb_chunks(CH, RPT)

_mesh = functools.partial(
    plsc.VectorSubcoreMesh, core_axis_name="c", subcore_axis_name="s")

_SC_PARAMS = pltpu.CompilerParams(use_tc_tiling_on_sc=False)


@functools.partial(
    pl.kernel,
    mesh=_mesh(),
    out_type=(
        jax.ShapeDtypeStruct((NPAD, D), jnp.float32),
        jax.ShapeDtypeStruct((NC * NPAD, HALF), jnp.float32),
    ),
    compiler_params=_SC_PARAMS,
    scratch_types=[
        pltpu.VMEM((CH,), jnp.int32),
        pltpu.VMEM((CH,), jnp.int32),
        pltpu.VMEM((CH,), jnp.int32),
        pltpu.VMEM((CH,), jnp.int32),
        pltpu.VMEM((CH,), jnp.int32),
        pltpu.VMEM((CH,), jnp.int32),
        pltpu.VMEM((CH,), jnp.int32),
        pltpu.VMEM((CH,), jnp.int32),
        pltpu.VMEM((CH, HALF), jnp.float32),
        pltpu.VMEM((CH, HALF), jnp.float32),
        pltpu.VMEM((CH,), jnp.float32),
        pltpu.VMEM((RPT,), jnp.float32),
        pltpu.VMEM_SHARED((NPAD, HALF), jnp.float32),
        pltpu.VMEM_SHARED((NPAD,), jnp.float32),
        pltpu.SemaphoreType.DMA,
        pltpu.SemaphoreType.DMA,
        pltpu.SemaphoreType.DMA,
        pltpu.SemaphoreType.DMA,
        pltpu.SemaphoreType.DMA,
        pltpu.SemaphoreType.DMA,
        pltpu.SemaphoreType.DMA,
        pltpu.SemaphoreType.DMA,
        pltpu.SemaphoreType.DMA,
    ],
)
def _gcn_call(table_hbm, src_hbm, dst_hbm, out_hbm, y_hbm,
              srcb0, srcb1, srcb2, srcb3, dstb0, dstb1, dstb2, dstb3,
              msg0, msg1, onesb, dinvb, acc, deg,
              isem0, isem1, isem2, isem3, gsem0, gsem1, ssem0, ssem1, ysem):
    c = lax.axis_index("c")
    s = lax.axis_index("s")
    srcb = (srcb0, srcb1, srcb2, srcb3)
    dstb = (dstb0, dstb1, dstb2, dstb3)
    msg = (msg0, msg1)
    isem = (isem0, isem1, isem2, isem3)
    gsem = (gsem0, gsem1)
    ssem = (ssem0, ssem1)

    def _fill(buf, val):
        def body(i, _):
            buf[pl.ds(i * 16, 16)] = jnp.full((16,), val, jnp.float32)
            return 0
        lax.fori_loop(0, CH // 16, body, 0)

    def _zero_msg(i, _):
        msg0[i >> 1, pl.ds((i & 1) * 16, 16)] = jnp.zeros((16,), jnp.float32)
        return 0

    lax.fori_loop(0, CH * 2, _zero_msg, 0)
    _fill(onesb, 0.0)
    for off, sz in _WB:
        pltpu.async_copy(msg0.at[pl.ds(0, sz)],
                         acc.at[pl.ds(s * RPT + off, sz)], gsem0)
        pltpu.async_copy(onesb.at[pl.ds(0, sz)],
                         deg.at[pl.ds(s * RPT + off, sz)], ysem)
    for off, sz in _WB:
        pltpu.make_async_copy(msg0.at[pl.ds(0, sz)],
                              acc.at[pl.ds(s * RPT + off, sz)], gsem0).wait()
        pltpu.make_async_copy(onesb.at[pl.ds(0, sz)],
                              deg.at[pl.ds(s * RPT + off, sz)], ysem).wait()
    _fill(onesb, 1.0)
    plsc.subcore_barrier()

    def _didx(i, q):
        ic = jnp.minimum(i, NCH - 1)
        pltpu.async_copy(dst_hbm.at[pl.ds(s * EPT + ic * CH, CH)],
                         dstb[q], isem[q])

    def _deg_step(i, q, first):
        if not first:
            pltpu.make_async_copy(onesb, deg.at[dstb[q]],
                                  ssem[q % 2]).wait()
        _didx(i + 2, (q + 2) % 4)
        pltpu.make_async_copy(dst_hbm.at[pl.ds(0, CH)], dstb[q],
                              isem[q]).wait()
        pltpu.async_copy(onesb, deg.at[dstb[q]], ssem[q % 2], add=True)

    _didx(0, 0)
    _didx(1, 1)
    _deg_step(0, 0, True)
    _deg_step(1, 1, True)
    _deg_step(2, 2, False)
    _deg_step(3, 3, False)

    def _deg_quad(g, _):
        for q in range(4):
            _deg_step(4 * g + q, q, False)
        return 0

    lax.fori_loop(1, NCH // 4, _deg_quad, 0)
    for i in (NCH - 2, NCH - 1):
        pltpu.make_async_copy(onesb, deg.at[dstb[i % 4]], ssem[i % 2]).wait()
    for i in (NCH, NCH + 1):
        pltpu.make_async_copy(dst_hbm.at[pl.ds(0, CH)], dstb[i % 4],
                              isem[i % 4]).wait()
    plsc.subcore_barrier()

    pltpu.sync_copy(deg.at[pl.ds(s * RPT, RPT)], dinvb)

    def _rsqrt16(i, _):
        v = dinvb[pl.ds(i * 16, 16)]
        bits = lax.bitcast_convert_type(v, jnp.int32)
        g = lax.bitcast_convert_type(
            jnp.int32(0x5F3759DF) - lax.shift_right_logical(bits, 1),
            jnp.float32)
        h = v * 0.5
        g = g * (1.5 - h * g * g)
        g = g * (1.5 - h * g * g)
        g = g * (1.5 - h * g * g)
        dinvb[pl.ds(i * 16, 16)] = jnp.where(v > 0, g, 0.0)
        return 0

    lax.fori_loop(0, RPT // 16, _rsqrt16, 0)

    def _scale_rows(buf_in, buf_out, row0, nrows, sq):
        def body(i, _):
            dvec = dinvb[pl.ds(row0 + i * 16, 16)]
            if sq:
                dvec = dvec * dvec
            for j in range(16):
                r = i * 16 + j
                dv = jnp.full((16,), dvec[j], jnp.float32)
                buf_out[r, pl.ds(0, 16)] = buf_in[r, pl.ds(0, 16)] * dv
                buf_out[r, pl.ds(16, 16)] = buf_in[r, pl.ds(16, 16)] * dv
            return 0
        lax.fori_loop(0, nrows // 16, body, 0)

    def _sub_scaled(row0, nrows):
        def body(i, _):
            dvec = dinvb[pl.ds(row0 + i * 16, 16)]
            dd = dvec * dvec
            for j in range(16):
                r = i * 16 + j
                dv = jnp.full((16,), dd[j], jnp.float32)
                msg0[r, pl.ds(0, 16)] = (
                    msg0[r, pl.ds(0, 16)] * dv - msg1[r, pl.ds(0, 16)])
                msg0[r, pl.ds(16, 16)] = (
                    msg0[r, pl.ds(16, 16)] * dv - msg1[r, pl.ds(16, 16)])
            return 0
        lax.fori_loop(0, nrows // 16, body, 0)

    def _final_rows(row0, nrows):
        def body(i, _):
            dvec = dinvb[pl.ds(row0 + i * 16, 16)] * 0.25
            for j in range(16):
                r = i * 16 + j
                dv = jnp.full((16,), dvec[j], jnp.float32)
                msg0[r, pl.ds(0, 16)] = (
                    msg1[r, pl.ds(0, 16)] * 0.25 + msg0[r, pl.ds(0, 16)] * dv)
                msg0[r, pl.ds(16, 16)] = (
                    msg1[r, pl.ds(16, 16)] * 0.25
                    + msg0[r, pl.ds(16, 16)] * dv)
            return 0
        lax.fori_loop(0, nrows // 16, body, 0)

    def _table_read(r0, sz, dst):
        return pltpu.async_copy(
            table_hbm.at[pl.ds(r0, sz), pl.ds(c * HALF, HALF)],
            dst.at[pl.ds(0, sz)], ysem)

    for off, sz in _WB:
        r0 = s * RPT + off
        _table_read(r0, sz, msg0).wait()
        _scale_rows(msg0, msg1, off, sz, False)
        pltpu.async_copy(msg1.at[pl.ds(0, sz)],
                         y_hbm.at[pl.ds(c * NPAD + r0, sz)], gsem0).wait()
    plsc.subcore_barrier()

    def _eidx(i, q):
        ic = jnp.minimum(i, NCH - 1)
        base = s * EPT + ic * CH
        pltpu.async_copy(src_hbm.at[pl.ds(c * EPAD + base, CH)],
                         srcb[q], isem[q])
        pltpu.async_copy(dst_hbm.at[pl.ds(base, CH)], dstb[q], isem[q])

    def _eidx_wait(q):
        pltpu.make_async_copy(src_hbm.at[pl.ds(0, CH)], srcb[q],
                              isem[q]).wait()
        pltpu.make_async_copy(dst_hbm.at[pl.ds(0, CH)], dstb[q],
                              isem[q]).wait()

    def _front(i, q, first):
        if not first:
            pltpu.make_async_copy(msg[q % 2], acc.at[dstb[q]],
                                  ssem[q % 2]).wait()
        _eidx(i + 2, (q + 2) % 4)
        _eidx_wait(q)
        pltpu.async_copy(y_hbm.at[srcb[q]], msg[q % 2], gsem[q % 2])

    def _back(q):
        pltpu.make_async_copy(y_hbm.at[srcb[q]], msg[q % 2],
                              gsem[q % 2]).wait()
        pltpu.async_copy(msg[q % 2], acc.at[dstb[q]], ssem[q % 2],
                         add=True)

    for layer in range(NLAYERS):
        _eidx(0, 0)
        _eidx(1, 1)
        _front(0, 0, True)
        _front(1, 1, True)
        _back(0)
        _back(1)
        _front(2, 2, False)
        _front(3, 3, False)
        _back(2)
        _back(3)

        def _quad(g, _):
            for q in range(4):
                _front(4 * g + q, q, False)
                _back(q)
            return 0

        lax.fori_loop(1, NCH // 4, _quad, 0)
        for i in (NCH - 2, NCH - 1):
            pltpu.make_async_copy(msg[i % 2], acc.at[dstb[i % 4]],
                                  ssem[i % 2]).wait()
        for i in (NCH, NCH + 1):
            _eidx_wait(i % 4)
        plsc.subcore_barrier()

        if layer == 0:
            for k, (off, sz) in enumerate(_WB):
                r0 = s * RPT + off
                pltpu.async_copy(acc.at[pl.ds(r0, sz)], msg0.at[pl.ds(0, sz)],
                                 gsem0).wait()
                if k > 0:
                    poff, psz = _WB[k - 1]
                    pltpu.make_async_copy(
                        msg1.at[pl.ds(0, psz)],
                        y_hbm.at[pl.ds(c * NPAD + s * RPT + poff, psz)],
                        ysem).wait()
                _scale_rows(msg0, msg1, off, sz, True)
                pltpu.async_copy(
                    msg1.at[pl.ds(0, sz)],
                    y_hbm.at[pl.ds(c * NPAD + r0, sz)], ysem)
            loff, lsz = _WB[-1]
            pltpu.make_async_copy(
                msg1.at[pl.ds(0, lsz)],
                y_hbm.at[pl.ds(c * NPAD + s * RPT + loff, lsz)], ysem).wait()
            plsc.subcore_barrier()
        elif layer == 1:
            for k, (off, sz) in enumerate(_WB):
                r0 = s * RPT + off
                if k > 0:
                    poff, psz = _WB[k - 1]
                    pltpu.make_async_copy(
                        msg0.at[pl.ds(0, psz)],
                        y_hbm.at[pl.ds(c * NPAD + s * RPT + poff, psz)],
                        ssem0).wait()
                ca = pltpu.async_copy(acc.at[pl.ds(r0, sz)],
                                      msg0.at[pl.ds(0, sz)], gsem0)
                cy = pltpu.async_copy(y_hbm.at[pl.ds(c * NPAD + r0, sz)],
                                      msg1.at[pl.ds(0, sz)], ysem)
                ca.wait()
                cy.wait()
                _sub_scaled(off, sz)
                pltpu.async_copy(
                    msg0.at[pl.ds(0, sz)],
                    y_hbm.at[pl.ds(c * NPAD + r0, sz)], ssem0)
            loff, lsz = _WB[-1]
            pltpu.make_async_copy(
                msg0.at[pl.ds(0, lsz)],
                y_hbm.at[pl.ds(c * NPAD + s * RPT + loff, lsz)], ssem0).wait()
            plsc.subcore_barrier()
        else:
            for k, (off, sz) in enumerate(_WB):
                r0 = s * RPT + off
                if k > 0:
                    poff, psz = _WB[k - 1]
                    pltpu.make_async_copy(
                        msg0.at[pl.ds(0, psz)],
                        out_hbm.at[pl.ds(s * RPT + poff, psz),
                                   pl.ds(c * HALF, HALF)], ssem0).wait()
                ca = pltpu.async_copy(acc.at[pl.ds(r0, sz)],
                                      msg0.at[pl.ds(0, sz)], gsem0)
                cx = _table_read(r0, sz, msg1)
                ca.wait()
                cx.wait()
                _final_rows(off, sz)
                pltpu.async_copy(
                    msg0.at[pl.ds(0, sz)],
                    out_hbm.at[pl.ds(r0, sz), pl.ds(c * HALF, HALF)], ssem0)
            loff, lsz = _WB[-1]
            pltpu.make_async_copy(
                msg0.at[pl.ds(0, lsz)],
                out_hbm.at[pl.ds(s * RPT + loff, lsz),
                           pl.ds(c * HALF, HALF)], ssem0).wait()


def kernel(edge_index, edge_attrs, table):
    del edge_attrs
    src = edge_index[0]
    dst = edge_index[1]

    pad_i = jnp.arange(EPAD - E, dtype=jnp.int32)
    src_p = jnp.concatenate([src, pad_i % N])
    dst_p = jnp.concatenate([dst, N + pad_i % NDUMMY])
    src2 = jnp.concatenate([src_p, src_p + NPAD])
    table_p = jnp.concatenate(
        [table, jnp.zeros((NPAD - N, D), jnp.float32)])

    final, _ = _gcn_call(table_p, src2, dst_p)
    return (table, final[:N])


# speedup vs baseline: 1.3777x; 1.1023x over previous
import functools

import jax
import jax.numpy as jnp
from jax import lax
from jax.experimental import pallas as pl
from jax.experimental.pallas import tpu as pltpu
from jax.experimental.pallas import tpu_sc as plsc

N = 50000
D = 64
E = 800000
HALF = 32
NS = 16
NC = 2

CH = 320
EPAD = 819200
EPT = EPAD // NS
NCH = EPT // CH
NPAD = 50176
RPT = NPAD // NS
NDUMMY = 128
NLAYERS = 3


def _wb_chunks(chunk, total):
    return tuple((o, min(chunk, total - o)) for o in range(0, total, chunk))


_WB = _wb_chunks(CH, RPT)

_mesh = functools.partial(
    plsc.VectorSubcoreMesh, core_axis_name="c", subcore_axis_name="s")

_SC_PARAMS = pltpu.CompilerParams(use_tc_tiling_on_sc=False)


@functools.partial(
    pl.kernel,
    mesh=_mesh(),
    out_type=(
        jax.ShapeDtypeStruct((NPAD, D), jnp.float32),
        jax.ShapeDtypeStruct((NC * NPAD, HALF), jnp.float32),
    ),
    compiler_params=_SC_PARAMS,
    scratch_types=[
        pltpu.VMEM((CH,), jnp.int32),
        pltpu.VMEM((CH,), jnp.int32),
        pltpu.VMEM((CH,), jnp.int32),
        pltpu.VMEM((CH,), jnp.int32),
        pltpu.VMEM((CH,), jnp.int32),
        pltpu.VMEM((CH,), jnp.int32),
        pltpu.VMEM((CH,), jnp.int32),
        pltpu.VMEM((CH,), jnp.int32),
        pltpu.VMEM((CH, HALF), jnp.float32),
        pltpu.VMEM((CH, HALF), jnp.float32),
        pltpu.VMEM((CH,), jnp.float32),
        pltpu.VMEM((RPT,), jnp.float32),
        pltpu.VMEM_SHARED((NPAD, HALF), jnp.float32),
        pltpu.VMEM_SHARED((NPAD,), jnp.float32),
        pltpu.SemaphoreType.DMA,
        pltpu.SemaphoreType.DMA,
        pltpu.SemaphoreType.DMA,
        pltpu.SemaphoreType.DMA,
        pltpu.SemaphoreType.DMA,
        pltpu.SemaphoreType.DMA,
        pltpu.SemaphoreType.DMA,
        pltpu.SemaphoreType.DMA,
        pltpu.SemaphoreType.DMA,
    ],
)
def _gcn_call(table_hbm, src_hbm, dst_hbm, out_hbm, y_hbm,
              srcb0, srcb1, srcb2, srcb3, dstb0, dstb1, dstb2, dstb3,
              msg0, msg1, onesb, dinvb, acc, deg,
              isem0, isem1, isem2, isem3, gsem0, gsem1, ssem0, ssem1, ysem):
    c = lax.axis_index("c")
    s = lax.axis_index("s")
    srcb = (srcb0, srcb1, srcb2, srcb3)
    dstb = (dstb0, dstb1, dstb2, dstb3)
    msg = (msg0, msg1)
    isem = (isem0, isem1, isem2, isem3)
    gsem = (gsem0, gsem1)
    ssem = (ssem0, ssem1)

    def _fill(buf, val):
        def body(i, _):
            buf[pl.ds(i * 16, 16)] = jnp.full((16,), val, jnp.float32)
            return 0
        lax.fori_loop(0, CH // 16, body, 0)

    def _zero_msg(i, _):
        msg0[i >> 1, pl.ds((i & 1) * 16, 16)] = jnp.zeros((16,), jnp.float32)
        return 0

    lax.fori_loop(0, CH * 2, _zero_msg, 0)
    _fill(onesb, 0.0)
    for off, sz in _WB:
        pltpu.async_copy(msg0.at[pl.ds(0, sz)],
                         acc.at[pl.ds(s * RPT + off, sz)], gsem0)
        pltpu.async_copy(onesb.at[pl.ds(0, sz)],
                         deg.at[pl.ds(s * RPT + off, sz)], ysem)
    for off, sz in _WB:
        pltpu.make_async_copy(msg0.at[pl.ds(0, sz)],
                              acc.at[pl.ds(s * RPT + off, sz)], gsem0).wait()
        pltpu.make_async_copy(onesb.at[pl.ds(0, sz)],
                              deg.at[pl.ds(s * RPT + off, sz)], ysem).wait()
    _fill(onesb, 1.0)
    plsc.subcore_barrier()

    def _didx(i, q):
        ic = jnp.minimum(i, NCH - 1)
        pltpu.async_copy(dst_hbm.at[pl.ds(s * EPT + ic * CH, CH)],
                         dstb[q], isem[q])

    def _deg_step(i, q, first):
        if not first:
            pltpu.make_async_copy(onesb, deg.at[dstb[q]],
                                  ssem[q % 2]).wait()
        _didx(i + 2, (q + 2) % 4)
        pltpu.make_async_copy(dst_hbm.at[pl.ds(0, CH)], dstb[q],
                              isem[q]).wait()
        pltpu.async_copy(onesb, deg.at[dstb[q]], ssem[q % 2], add=True)

    _didx(0, 0)
    _didx(1, 1)
    _deg_step(0, 0, True)
    _deg_step(1, 1, True)
    _deg_step(2, 2, False)
    _deg_step(3, 3, False)

    def _deg_quad(g, _):
        for q in range(4):
            _deg_step(4 * g + q, q, False)
        return 0

    lax.fori_loop(1, NCH // 4, _deg_quad, 0)
    for i in (NCH - 2, NCH - 1):
        pltpu.make_async_copy(onesb, deg.at[dstb[i % 4]], ssem[i % 2]).wait()
    for i in (NCH, NCH + 1):
        pltpu.make_async_copy(dst_hbm.at[pl.ds(0, CH)], dstb[i % 4],
                              isem[i % 4]).wait()
    plsc.subcore_barrier()

    pltpu.sync_copy(deg.at[pl.ds(s * RPT, RPT)], dinvb)

    def _rsqrt16(i, _):
        v = dinvb[pl.ds(i * 16, 16)]
        bits = lax.bitcast_convert_type(v, jnp.int32)
        g = lax.bitcast_convert_type(
            jnp.int32(0x5F3759DF) - lax.shift_right_logical(bits, 1),
            jnp.float32)
        h = v * 0.5
        g = g * (1.5 - h * g * g)
        g = g * (1.5 - h * g * g)
        g = g * (1.5 - h * g * g)
        dinvb[pl.ds(i * 16, 16)] = jnp.where(v > 0, g, 0.0)
        return 0

    lax.fori_loop(0, RPT // 16, _rsqrt16, 0)

    def _scale_rows(buf_in, buf_out, row0, nrows, sq):
        def body(i, _):
            dvec = dinvb[pl.ds(row0 + i * 16, 16)]
            if sq:
                dvec = dvec * dvec
            for j in range(16):
                r = i * 16 + j
                dv = jnp.full((16,), dvec[j], jnp.float32)
                buf_out[r, pl.ds(0, 16)] = buf_in[r, pl.ds(0, 16)] * dv
                buf_out[r, pl.ds(16, 16)] = buf_in[r, pl.ds(16, 16)] * dv
            return 0
        lax.fori_loop(0, nrows // 16, body, 0)

    def _sub_scaled(row0, nrows):
        def body(i, _):
            dvec = dinvb[pl.ds(row0 + i * 16, 16)]
            dd = dvec * dvec
            for j in range(16):
                r = i * 16 + j
                dv = jnp.full((16,), dd[j], jnp.float32)
                msg0[r, pl.ds(0, 16)] = (
                    msg0[r, pl.ds(0, 16)] * dv - msg1[r, pl.ds(0, 16)])
                msg0[r, pl.ds(16, 16)] = (
                    msg0[r, pl.ds(16, 16)] * dv - msg1[r, pl.ds(16, 16)])
            return 0
        lax.fori_loop(0, nrows // 16, body, 0)

    def _final_rows(row0, nrows):
        def body(i, _):
            dvec = dinvb[pl.ds(row0 + i * 16, 16)] * 0.25
            for j in range(16):
                r = i * 16 + j
                dv = jnp.full((16,), dvec[j], jnp.float32)
                msg0[r, pl.ds(0, 16)] = (
                    msg1[r, pl.ds(0, 16)] * 0.25 + msg0[r, pl.ds(0, 16)] * dv)
                msg0[r, pl.ds(16, 16)] = (
                    msg1[r, pl.ds(16, 16)] * 0.25
                    + msg0[r, pl.ds(16, 16)] * dv)
            return 0
        lax.fori_loop(0, nrows // 16, body, 0)

    def _table_read(r0, sz, dst):
        return pltpu.async_copy(
            table_hbm.at[pl.ds(r0, sz), pl.ds(c * HALF, HALF)],
            dst.at[pl.ds(0, sz)], ysem)

    for off, sz in _WB:
        r0 = s * RPT + off
        _table_read(r0, sz, msg0).wait()
        _scale_rows(msg0, msg1, off, sz, False)
        pltpu.async_copy(msg1.at[pl.ds(0, sz)],
                         y_hbm.at[pl.ds(c * NPAD + r0, sz)], gsem0).wait()
    plsc.subcore_barrier()

    def _eidx(i, q):
        ic = jnp.minimum(i, NCH - 1)
        base = s * EPT + ic * CH
        pltpu.async_copy(src_hbm.at[pl.ds(c * EPAD + base, CH)],
                         srcb[q], isem[q])
        pltpu.async_copy(dst_hbm.at[pl.ds(base, CH)], dstb[q], isem[q])

    def _eidx_wait(q):
        pltpu.make_async_copy(src_hbm.at[pl.ds(0, CH)], srcb[q],
                              isem[q]).wait()
        pltpu.make_async_copy(dst_hbm.at[pl.ds(0, CH)], dstb[q],
                              isem[q]).wait()

    def _front(i, q, first):
        if not first:
            pltpu.make_async_copy(msg[q % 2], acc.at[dstb[q]],
                                  ssem[q % 2]).wait()
        _eidx(i + 2, (q + 2) % 4)
        _eidx_wait(q)
        pltpu.async_copy(y_hbm.at[srcb[q]], msg[q % 2], gsem[q % 2])

    def _back(q):
        pltpu.make_async_copy(y_hbm.at[srcb[q]], msg[q % 2],
                              gsem[q % 2]).wait()
        pltpu.async_copy(msg[q % 2], acc.at[dstb[q]], ssem[q % 2],
                         add=True)

    for layer in range(NLAYERS):
        _eidx(0, 0)
        _eidx(1, 1)
        _front(0, 0, True)
        _front(1, 1, True)
        _back(0)
        _back(1)
        _front(2, 2, False)
        _front(3, 3, False)
        _back(2)
        _back(3)

        def _quad(g, _):
            for q in range(4):
                _front(4 * g + q, q, False)
                _back(q)
            return 0

        lax.fori_loop(1, NCH // 4, _quad, 0)
        for i in (NCH - 2, NCH - 1):
            pltpu.make_async_copy(msg[i % 2], acc.at[dstb[i % 4]],
                                  ssem[i % 2]).wait()
        for i in (NCH, NCH + 1):
            _eidx_wait(i % 4)
        plsc.subcore_barrier()

        if layer == 0:
            for k, (off, sz) in enumerate(_WB):
                r0 = s * RPT + off
                pltpu.async_copy(acc.at[pl.ds(r0, sz)], msg0.at[pl.ds(0, sz)],
                                 gsem0).wait()
                if k > 0:
                    poff, psz = _WB[k - 1]
                    pltpu.make_async_copy(
                        msg1.at[pl.ds(0, psz)],
                        y_hbm.at[pl.ds(c * NPAD + s * RPT + poff, psz)],
                        ysem).wait()
                _scale_rows(msg0, msg1, off, sz, True)
                pltpu.async_copy(
                    msg1.at[pl.ds(0, sz)],
                    y_hbm.at[pl.ds(c * NPAD + r0, sz)], ysem)
            loff, lsz = _WB[-1]
            pltpu.make_async_copy(
                msg1.at[pl.ds(0, lsz)],
                y_hbm.at[pl.ds(c * NPAD + s * RPT + loff, lsz)], ysem).wait()
            plsc.subcore_barrier()
        elif layer == 1:
            for k, (off, sz) in enumerate(_WB):
                r0 = s * RPT + off
                if k > 0:
                    poff, psz = _WB[k - 1]
                    pltpu.make_async_copy(
                        msg0.at[pl.ds(0, psz)],
                        y_hbm.at[pl.ds(c * NPAD + s * RPT + poff, psz)],
                        ssem0).wait()
                ca = pltpu.async_copy(acc.at[pl.ds(r0, sz)],
                                      msg0.at[pl.ds(0, sz)], gsem0)
                cy = pltpu.async_copy(y_hbm.at[pl.ds(c * NPAD + r0, sz)],
                                      msg1.at[pl.ds(0, sz)], ysem)
                ca.wait()
                cy.wait()
                _sub_scaled(off, sz)
                pltpu.async_copy(
                    msg0.at[pl.ds(0, sz)],
                    y_hbm.at[pl.ds(c * NPAD + r0, sz)], ssem0)
            loff, lsz = _WB[-1]
            pltpu.make_async_copy(
                msg0.at[pl.ds(0, lsz)],
                y_hbm.at[pl.ds(c * NPAD + s * RPT + loff, lsz)], ssem0).wait()
            plsc.subcore_barrier()
        else:
            for k, (off, sz) in enumerate(_WB):
                r0 = s * RPT + off
                if k > 0:
                    poff, psz = _WB[k - 1]
                    pltpu.make_async_copy(
                        msg0.at[pl.ds(0, psz)],
                        out_hbm.at[pl.ds(s * RPT + poff, psz),
                                   pl.ds(c * HALF, HALF)], ssem0).wait()
                ca = pltpu.async_copy(acc.at[pl.ds(r0, sz)],
                                      msg0.at[pl.ds(0, sz)], gsem0)
                cx = _table_read(r0, sz, msg1)
                ca.wait()
                cx.wait()
                _final_rows(off, sz)
                pltpu.async_copy(
                    msg0.at[pl.ds(0, sz)],
                    out_hbm.at[pl.ds(r0, sz), pl.ds(c * HALF, HALF)], ssem0)
            loff, lsz = _WB[-1]
            pltpu.make_async_copy(
                msg0.at[pl.ds(0, lsz)],
                out_hbm.at[pl.ds(s * RPT + loff, lsz),
                           pl.ds(c * HALF, HALF)], ssem0).wait()


def kernel(edge_index, edge_attrs, table):
    del edge_attrs
    src = edge_index[0]
    dst = edge_index[1]

    pad_i = jnp.arange(EPAD - E, dtype=jnp.int32)
    src_p = jnp.concatenate([src, pad_i % N])
    dst_p = jnp.concatenate([dst, N + pad_i % NDUMMY])
    src2 = jnp.concatenate([src_p, src_p + NPAD])
    table_p = jnp.concatenate(
        [table, jnp.zeros((NPAD - N, D), jnp.float32)])

    final, _ = _gcn_call(table_p, src2, dst_p)
    return (table, final[:N])
